# baseline (speedup 1.0000x reference)
import jax
import jax.numpy as jnp
from jax.experimental import pallas as pl
from jax.experimental.pallas import tpu as pltpu

def _make_body(B, S, D, V, R):
    H = S // 2
    N = 2 * B

    def body(x_ref, table_ref, out_ref, buf, pe_ref, in_sem, out_sem):
        def in_copy(c):
            return pltpu.make_async_copy(
                x_ref.at[c // 2, pl.ds((c % 2) * H, H), :],
                buf.at[c],
                in_sem.at[c],
            )

        def out_copy(c):
            return pltpu.make_async_copy(
                buf.at[c],
                out_ref.at[c // 2, pl.ds((c % 2) * H, H), :],
                out_sem.at[c],
            )

        for c in range(N):
            in_copy(c).start()

        i = jax.lax.broadcasted_iota(jnp.int32, (S, V), 0)
        k = jax.lax.broadcasted_iota(jnp.int32, (S, V), 1)
        rel = k - R
        counts = jnp.logical_and(rel >= -i, rel <= S - 1 - i).astype(jnp.float32)
        n_lo = jnp.maximum(i - (R - 1), 0).astype(jnp.float32)
        n_hi = jnp.maximum(S - i - R, 0).astype(jnp.float32)
        counts = jnp.where(k == 0, n_lo, counts)
        counts = jnp.where(k == V - 1, n_hi, counts)
        pe_ref[...] = jnp.dot(
            counts, table_ref[...], preferred_element_type=jnp.float32
        ) * (1.0 / S)

        for c in range(N):
            in_copy(c).wait()
            off = (c % 2) * H
            buf[c, :, :] = buf[c, :, :] + pe_ref[off : off + H, :]
            out_copy(c).start()
        for c in range(N):
            out_copy(c).wait()

    return body


def kernel(x, table):
    B, S, D = x.shape
    V, _ = table.shape
    R = (V - 1) // 2
    return pl.pallas_call(
        _make_body(B, S, D, V, R),
        in_specs=[
            pl.BlockSpec(memory_space=pl.ANY),
            pl.BlockSpec(memory_space=pltpu.MemorySpace.VMEM),
        ],
        out_specs=pl.BlockSpec(memory_space=pl.ANY),
        out_shape=jax.ShapeDtypeStruct((B, S, D), x.dtype),
        scratch_shapes=[
            pltpu.VMEM((2 * B, S // 2, D), jnp.float32),
            pltpu.VMEM((S, D), jnp.float32),
            pltpu.SemaphoreType.DMA((2 * B,)),
            pltpu.SemaphoreType.DMA((2 * B,)),
        ],
    )(x, table)


# speedup vs baseline: 1.0063x; 1.0063x over previous
import jax
import jax.numpy as jnp
from jax.experimental import pallas as pl
from jax.experimental.pallas import tpu as pltpu

def _make_body(B, S, D, V, R):
    def body(x_ref, table_ref, out_ref, buf, pe_ref, in_sem, out_sem):
        def in_copy(c):
            return pltpu.make_async_copy(x_ref.at[c], buf.at[c], in_sem.at[c])

        def out_copy(c):
            return pltpu.make_async_copy(buf.at[c], out_ref.at[c], out_sem.at[c])

        for c in range(B):
            in_copy(c).start()

        i = jax.lax.broadcasted_iota(jnp.int32, (S, V), 0)
        k = jax.lax.broadcasted_iota(jnp.int32, (S, V), 1)
        rel = k - R
        counts = jnp.logical_and(rel >= -i, rel <= S - 1 - i).astype(jnp.float32)
        n_lo = jnp.maximum(i - (R - 1), 0).astype(jnp.float32)
        n_hi = jnp.maximum(S - i - R, 0).astype(jnp.float32)
        counts = jnp.where(k == 0, n_lo, counts)
        counts = jnp.where(k == V - 1, n_hi, counts)
        pe_ref[...] = jnp.dot(
            counts, table_ref[...], preferred_element_type=jnp.float32
        ) * (1.0 / S)

        for c in range(B):
            in_copy(c).wait()
            buf[c, :, :] = buf[c, :, :] + pe_ref[...]
            out_copy(c).start()
        for c in range(B):
            out_copy(c).wait()

    return body


def kernel(x, table):
    B, S, D = x.shape
    V, _ = table.shape
    R = (V - 1) // 2
    return pl.pallas_call(
        _make_body(B, S, D, V, R),
        in_specs=[
            pl.BlockSpec(memory_space=pl.ANY),
            pl.BlockSpec(memory_space=pltpu.MemorySpace.VMEM),
        ],
        out_specs=pl.BlockSpec(memory_space=pl.ANY),
        out_shape=jax.ShapeDtypeStruct((B, S, D), x.dtype),
        scratch_shapes=[
            pltpu.VMEM((B, S, D), jnp.float32),
            pltpu.VMEM((S, D), jnp.float32),
            pltpu.SemaphoreType.DMA((B,)),
            pltpu.SemaphoreType.DMA((B,)),
        ],
    )(x, table)
